# Initial kernel scaffold; baseline (speedup 1.0000x reference)
import jax
import jax.numpy as jnp
from jax.experimental import pallas as pl


def kernel(x, edge_index, W0, b0, W1, b1, W2, b2, Wjk, bjk):
    raise NotImplementedError("write your pallas kernel here")


# speedup vs baseline: 13.8130x; 13.8130x over previous
import functools

import jax
import jax.numpy as jnp
from jax import lax
from jax.experimental import pallas as pl
from jax.experimental.pallas import tpu as pltpu
from jax.experimental.pallas import tpu_sc as plsc

NC = 2
NS = 16
EK = 128


def _spmm_sc(g, src, dst, do_gather=True):
    N, D = g.shape
    E = src.shape[0]
    EC = E // NC
    C = EC // EK
    CPT = pl.cdiv(C, NS)

    G = N // 8
    base_g = G // NS
    rem = G - base_g * NS

    def _row_ranges(sid, fn):
        if rem:
            @pl.when(sid < rem)
            def _():
                fn(sid * (base_g + 1) * 8, (base_g + 1) * 8)

            @pl.when(sid >= rem)
            def _():
                fn((rem * (base_g + 1) + (sid - rem) * base_g) * 8, base_g * 8)
        else:
            fn(sid * base_g * 8, base_g * 8)

    mesh = plsc.VectorSubcoreMesh(core_axis_name="c", subcore_axis_name="s")

    @functools.partial(
        pl.kernel,
        mesh=mesh,
        out_type=jax.ShapeDtypeStruct((NC, N, D), jnp.float32),
        scratch_types=[
            pltpu.VMEM((EK,), jnp.int32),
            pltpu.VMEM((EK,), jnp.int32),
            pltpu.VMEM((EK, D), jnp.float32),
            pltpu.VMEM_SHARED((N, D), jnp.float32),
            pltpu.SemaphoreType.DMA,
        ],
    )
    def k(g_hbm, src_hbm, dst_hbm, out_hbm, src_v, dst_v, rows_v, acc, sem):
        cid = lax.axis_index("c")
        sid = lax.axis_index("s")
        _row_ranges(sid, lambda off, cnt: pltpu.sync_copy(
            g_hbm.at[pl.ds(off, cnt)], acc.at[pl.ds(off, cnt)]))
        if not do_gather:
            def fill(r, carry):
                for j in range(D // 16):
                    rows_v[r, pl.ds(j * 16, 16)] = jnp.full((16,), 1.0,
                                                            jnp.float32)
                return carry
            lax.fori_loop(0, EK, fill, 0)
        plsc.subcore_barrier()

        core_base = cid * EC

        def body(i, carry):
            chunk = sid + i * NS

            @pl.when(chunk < C)
            def _():
                base = core_base + chunk * EK
                pltpu.sync_copy(dst_hbm.at[pl.ds(base, EK)], dst_v)
                if do_gather:
                    pltpu.sync_copy(src_hbm.at[pl.ds(base, EK)], src_v)
                    pltpu.async_copy(g_hbm.at[src_v], rows_v, sem).wait()
                pltpu.sync_copy(rows_v, acc.at[dst_v], add=True)

            return carry

        lax.fori_loop(0, CPT, body, 0)
        plsc.subcore_barrier()
        _row_ranges(sid, lambda off, cnt: pltpu.sync_copy(
            acc.at[pl.ds(off, cnt)], out_hbm.at[cid, pl.ds(off, cnt)]))

    return k(g, src, dst)


_BN = 1000


def _tc_first(deg_p, x, W0):
    N, D = x.shape
    H = W0.shape[1]

    def body(dp_ref, x_ref, w_ref, g_ref, dinvb_ref):
        deg = dp_ref[0, :, 0:1] + dp_ref[1, :, 0:1] - 1.0
        dinv = lax.rsqrt(deg)
        hw = jnp.dot(x_ref[...], w_ref[...], preferred_element_type=jnp.float32)
        g_ref[...] = hw * dinv
        dinvb_ref[...] = jnp.broadcast_to(dinv, (_BN, H))

    return pl.pallas_call(
        body,
        grid=(N // _BN,),
        in_specs=[
            pl.BlockSpec((NC, _BN, 16), lambda i: (0, i, 0)),
            pl.BlockSpec((_BN, D), lambda i: (i, 0)),
            pl.BlockSpec((D, H), lambda i: (0, 0)),
        ],
        out_specs=[
            pl.BlockSpec((_BN, H), lambda i: (i, 0)),
            pl.BlockSpec((_BN, H), lambda i: (i, 0)),
        ],
        out_shape=[
            jax.ShapeDtypeStruct((N, H), jnp.float32),
            jax.ShapeDtypeStruct((N, H), jnp.float32),
        ],
    )(deg_p, x, W0)


def _tc_mid(sp, g_prev, dinvb, b, Wn):
    N, H = g_prev.shape

    def body(sp_ref, gp_ref, dinvb_ref, b_ref, w_ref, h_ref, g_ref):
        dinvb = dinvb_ref[...]
        s = sp_ref[0] + sp_ref[1]
        h = jnp.maximum(dinvb * (s - gp_ref[...]) + b_ref[...], 0.0)
        h_ref[...] = h
        g_ref[...] = jnp.dot(h, w_ref[...], preferred_element_type=jnp.float32) * dinvb

    return pl.pallas_call(
        body,
        grid=(N // _BN,),
        in_specs=[
            pl.BlockSpec((NC, _BN, H), lambda i: (0, i, 0)),
            pl.BlockSpec((_BN, H), lambda i: (i, 0)),
            pl.BlockSpec((_BN, H), lambda i: (i, 0)),
            pl.BlockSpec((1, H), lambda i: (0, 0)),
            pl.BlockSpec((H, H), lambda i: (0, 0)),
        ],
        out_specs=[
            pl.BlockSpec((_BN, H), lambda i: (i, 0)),
            pl.BlockSpec((_BN, H), lambda i: (i, 0)),
        ],
        out_shape=[
            jax.ShapeDtypeStruct((N, H), jnp.float32),
            jax.ShapeDtypeStruct((N, H), jnp.float32),
        ],
    )(sp, g_prev, dinvb, b, Wn)


def _tc_last(sp, g_prev, dinvb, b, h1, h2, Wjk, bjk):
    N, H = g_prev.shape
    O = Wjk.shape[1]

    def body(sp_ref, gp_ref, dinvb_ref, b_ref, h1_ref, h2_ref, wjk_ref,
             bjk_ref, out_ref):
        dinvb = dinvb_ref[...]
        s = sp_ref[0] + sp_ref[1]
        h3 = jnp.maximum(dinvb * (s - gp_ref[...]) + b_ref[...], 0.0)
        jk = (h1_ref[...] + h2_ref[...] + h3) * (1.0 / 3.0)
        out_ref[...] = (
            jnp.dot(jk, wjk_ref[...], preferred_element_type=jnp.float32)
            + bjk_ref[...]
        )

    return pl.pallas_call(
        body,
        grid=(N // _BN,),
        in_specs=[
            pl.BlockSpec((NC, _BN, H), lambda i: (0, i, 0)),
            pl.BlockSpec((_BN, H), lambda i: (i, 0)),
            pl.BlockSpec((_BN, H), lambda i: (i, 0)),
            pl.BlockSpec((1, H), lambda i: (0, 0)),
            pl.BlockSpec((_BN, H), lambda i: (i, 0)),
            pl.BlockSpec((_BN, H), lambda i: (i, 0)),
            pl.BlockSpec((H, O), lambda i: (0, 0)),
            pl.BlockSpec((1, O), lambda i: (0, 0)),
        ],
        out_specs=pl.BlockSpec((_BN, O), lambda i: (i, 0)),
        out_shape=jax.ShapeDtypeStruct((N, O), jnp.float32),
    )(sp, g_prev, dinvb, b, h1, h2, Wjk, bjk)


def kernel(x, edge_index, W0, b0, W1, b1, W2, b2, Wjk, bjk):
    N = x.shape[0]
    ei = edge_index.astype(jnp.int32)
    src, dst = ei[0], ei[1]

    ones16 = jnp.ones((N, 16), jnp.float32)
    deg_p = _spmm_sc(ones16, src, dst, do_gather=False)

    g0, dinvb = _tc_first(deg_p, x, W0)
    sp0 = _spmm_sc(g0, src, dst)
    h1, g1 = _tc_mid(sp0, g0, dinvb, b0.reshape(1, -1), W1)
    sp1 = _spmm_sc(g1, src, dst)
    h2, g2 = _tc_mid(sp1, g1, dinvb, b1.reshape(1, -1), W2)
    sp2 = _spmm_sc(g2, src, dst)
    return _tc_last(sp2, g2, dinvb, b2.reshape(1, -1), h1, h2, Wjk,
                    bjk.reshape(1, -1))
